# Initial kernel scaffold; baseline (speedup 1.0000x reference)
import jax
import jax.numpy as jnp
from jax.experimental import pallas as pl


def kernel(x, gate_weight):
    raise NotImplementedError("write your pallas kernel here")


# speedup vs baseline: 1.1787x; 1.1787x over previous
import jax
import jax.numpy as jnp
from jax.experimental import pallas as pl

_TOP_K = 8
_NUM_EXPERTS = 64
_BLOCK_M = 512


def _router_kernel(x_ref, w_ref, ts_ref, idx_ref):
    x = x_ref[...]
    w = w_ref[...]
    scores = jax.lax.dot_general(
        x, w, (((1,), (1,)), ((), ())), preferred_element_type=jnp.float32
    )
    s = jax.nn.sigmoid(scores)
    lane = jax.lax.broadcasted_iota(jnp.int32, s.shape, 1)
    vals = []
    idxs = []
    cur = s
    for _ in range(_TOP_K):
        m = jnp.max(cur, axis=1, keepdims=True)
        sel = jnp.min(
            jnp.where(cur == m, lane, _NUM_EXPERTS), axis=1, keepdims=True
        )
        vals.append(m)
        idxs.append(sel)
        cur = jnp.where(lane == sel, -jnp.inf, cur)
    tv = jnp.concatenate(vals, axis=1)
    ti = jnp.concatenate(idxs, axis=1)
    denom = jnp.sum(tv, axis=1, keepdims=True) + 1e-20
    ts_ref[...] = tv / denom
    idx_ref[...] = ti


def kernel(x, gate_weight):
    n_tokens = x.shape[0]
    grid = (n_tokens // _BLOCK_M,)
    return pl.pallas_call(
        _router_kernel,
        grid=grid,
        in_specs=[
            pl.BlockSpec((_BLOCK_M, x.shape[1]), lambda i: (i, 0)),
            pl.BlockSpec(gate_weight.shape, lambda i: (0, 0)),
        ],
        out_specs=[
            pl.BlockSpec((_BLOCK_M, _TOP_K), lambda i: (i, 0)),
            pl.BlockSpec((_BLOCK_M, _TOP_K), lambda i: (i, 0)),
        ],
        out_shape=[
            jax.ShapeDtypeStruct((n_tokens, _TOP_K), jnp.float32),
            jax.ShapeDtypeStruct((n_tokens, _TOP_K), jnp.int32),
        ],
    )(x, gate_weight)


# speedup vs baseline: 1.3001x; 1.1030x over previous
import jax
import jax.numpy as jnp
from jax.experimental import pallas as pl

_TOP_K = 8
_NUM_EXPERTS = 64
_BLOCK_M = 512
_QBITS = 25
_QSCALE = float(2 ** _QBITS)


def _router_kernel(x_ref, w_ref, ts_ref, idx_ref):
    x = x_ref[...]
    w = w_ref[...]
    scores = jax.lax.dot_general(
        x, w, (((1,), (1,)), ((), ())), preferred_element_type=jnp.float32
    )
    s = jax.nn.sigmoid(scores)
    q = jnp.minimum((s * _QSCALE).astype(jnp.int32), (1 << _QBITS) - 1)
    lane = jax.lax.broadcasted_iota(jnp.int32, s.shape, 1)
    key = q * _NUM_EXPERTS + ((_NUM_EXPERTS - 1) - lane)
    vals = []
    cur = key
    for _ in range(_TOP_K):
        m = jnp.max(cur, axis=1, keepdims=True)
        vals.append(m)
        cur = jnp.where(cur == m, -1, cur)
    k8 = jnp.concatenate(vals, axis=1)
    idx = (_NUM_EXPERTS - 1) - (k8 & (_NUM_EXPERTS - 1))
    v = (k8 >> 6).astype(jnp.float32) * (1.0 / _QSCALE)
    denom = jnp.sum(v, axis=1, keepdims=True) + 1e-20
    ts_ref[...] = v / denom
    idx_ref[...] = idx


def kernel(x, gate_weight):
    n_tokens = x.shape[0]
    grid = (n_tokens // _BLOCK_M,)
    return pl.pallas_call(
        _router_kernel,
        grid=grid,
        in_specs=[
            pl.BlockSpec((_BLOCK_M, x.shape[1]), lambda i: (i, 0)),
            pl.BlockSpec(gate_weight.shape, lambda i: (0, 0)),
        ],
        out_specs=[
            pl.BlockSpec((_BLOCK_M, _TOP_K), lambda i: (i, 0)),
            pl.BlockSpec((_BLOCK_M, _TOP_K), lambda i: (i, 0)),
        ],
        out_shape=[
            jax.ShapeDtypeStruct((n_tokens, _TOP_K), jnp.float32),
            jax.ShapeDtypeStruct((n_tokens, _TOP_K), jnp.int32),
        ],
    )(x, gate_weight)


# speedup vs baseline: 1.4367x; 1.1051x over previous
import jax
import jax.numpy as jnp
from jax.experimental import pallas as pl

_TOP_K = 8
_NUM_EXPERTS = 64
_BLOCK_M = 512
_QBITS = 25
_QSCALE = float(2 ** _QBITS)


def _router_kernel(x_ref, w_ref, ts_ref, idx_ref):
    x = x_ref[...]
    w = w_ref[...]
    scores = jax.lax.dot_general(
        x, w, (((1,), (1,)), ((), ())), preferred_element_type=jnp.float32
    )
    s = jax.nn.sigmoid(scores)
    lane = jax.lax.broadcasted_iota(jnp.int32, s.shape, 1)
    sbits = jax.lax.bitcast_convert_type(s, jnp.int32)
    kbits = (sbits & ~(_NUM_EXPERTS - 1)) | ((_NUM_EXPERTS - 1) - lane)
    cur = jax.lax.bitcast_convert_type(kbits, jnp.float32)
    vals = []
    for _ in range(_TOP_K):
        m = jnp.max(cur, axis=1, keepdims=True)
        vals.append(m)
        cur = jnp.where(cur == m, -1.0, cur)
    k8 = jax.lax.bitcast_convert_type(jnp.concatenate(vals, axis=1), jnp.int32)
    idx = (_NUM_EXPERTS - 1) - (k8 & (_NUM_EXPERTS - 1))
    v = jax.lax.bitcast_convert_type(k8 & ~(_NUM_EXPERTS - 1), jnp.float32)
    denom = jnp.sum(v, axis=1, keepdims=True) + 1e-20
    ts_ref[...] = v / denom
    idx_ref[...] = idx


def kernel(x, gate_weight):
    n_tokens = x.shape[0]
    grid = (n_tokens // _BLOCK_M,)
    return pl.pallas_call(
        _router_kernel,
        grid=grid,
        in_specs=[
            pl.BlockSpec((_BLOCK_M, x.shape[1]), lambda i: (i, 0)),
            pl.BlockSpec(gate_weight.shape, lambda i: (0, 0)),
        ],
        out_specs=[
            pl.BlockSpec((_BLOCK_M, _TOP_K), lambda i: (i, 0)),
            pl.BlockSpec((_BLOCK_M, _TOP_K), lambda i: (i, 0)),
        ],
        out_shape=[
            jax.ShapeDtypeStruct((n_tokens, _TOP_K), jnp.float32),
            jax.ShapeDtypeStruct((n_tokens, _TOP_K), jnp.int32),
        ],
    )(x, gate_weight)


# speedup vs baseline: 1.5669x; 1.0906x over previous
import jax
import jax.numpy as jnp
from jax.experimental import pallas as pl

_TOP_K = 8
_NUM_EXPERTS = 64
_BLOCK_M = 1024
_QBITS = 25
_QSCALE = float(2 ** _QBITS)


def _router_kernel(x_ref, w_ref, ts_ref, idx_ref):
    x = x_ref[...]
    w = w_ref[...]
    scores = jax.lax.dot_general(
        x, w, (((1,), (1,)), ((), ())), preferred_element_type=jnp.float32
    )
    s = jax.nn.sigmoid(scores)
    lane = jax.lax.broadcasted_iota(jnp.int32, s.shape, 1)
    sbits = jax.lax.bitcast_convert_type(s, jnp.int32)
    kbits = (sbits & ~(_NUM_EXPERTS - 1)) | ((_NUM_EXPERTS - 1) - lane)
    cur = jax.lax.bitcast_convert_type(kbits, jnp.float32)
    vals = []
    for _ in range(_TOP_K):
        m = jnp.max(cur, axis=1, keepdims=True)
        vals.append(m)
        cur = jnp.where(cur == m, -1.0, cur)
    k8 = jax.lax.bitcast_convert_type(jnp.concatenate(vals, axis=1), jnp.int32)
    idx = (_NUM_EXPERTS - 1) - (k8 & (_NUM_EXPERTS - 1))
    v = jax.lax.bitcast_convert_type(k8 & ~(_NUM_EXPERTS - 1), jnp.float32)
    denom = jnp.sum(v, axis=1, keepdims=True) + 1e-20
    ts_ref[...] = v / denom
    idx_ref[...] = idx


def kernel(x, gate_weight):
    n_tokens = x.shape[0]
    grid = (n_tokens // _BLOCK_M,)
    return pl.pallas_call(
        _router_kernel,
        grid=grid,
        in_specs=[
            pl.BlockSpec((_BLOCK_M, x.shape[1]), lambda i: (i, 0)),
            pl.BlockSpec(gate_weight.shape, lambda i: (0, 0)),
        ],
        out_specs=[
            pl.BlockSpec((_BLOCK_M, _TOP_K), lambda i: (i, 0)),
            pl.BlockSpec((_BLOCK_M, _TOP_K), lambda i: (i, 0)),
        ],
        out_shape=[
            jax.ShapeDtypeStruct((n_tokens, _TOP_K), jnp.float32),
            jax.ShapeDtypeStruct((n_tokens, _TOP_K), jnp.int32),
        ],
    )(x, gate_weight)
